# baseline (speedup 1.0000x reference)
import jax
import jax.numpy as jnp
from jax.experimental import pallas as pl
from jax.experimental.pallas import tpu as pltpu


def _gat_fused(nf_ref, adj_ref, w_ref, al_ref, ar_ref, out_ref, att_ref,
               feat_ref):
    H, D = al_ref.shape
    jb = pl.program_id(1)
    Wj = adj_ref.shape[2]

    @pl.when(jb == 0)
    def _():
        feat_ref[...] = jnp.dot(nf_ref[0], w_ref[...],
                                preferred_element_type=jnp.float32)

    mask = adj_ref[0] > 0
    outs = []
    for h in range(H):
        feat_h = feat_ref[:, h * D:(h + 1) * D]
        fblk_h = feat_ref[pl.ds(jb * Wj, Wj), h * D:(h + 1) * D]
        al_h = al_ref[h, :].reshape(1, D)
        ar_h = ar_ref[h, :].reshape(1, D)
        el = jax.lax.dot_general(feat_h, al_h, (((1,), (1,)), ((), ())),
                                 preferred_element_type=jnp.float32)
        er = jax.lax.dot_general(ar_h, fblk_h, (((1,), (1,)), ((), ())),
                                 preferred_element_type=jnp.float32)
        s = el + er
        s = jnp.where(s >= 0.0, s, 0.2 * s)
        neg = jnp.where(mask, s, -1e30)
        m = jnp.max(neg, axis=0, keepdims=True)
        ex = jnp.where(mask, jnp.exp(neg - m), 0.0)
        denom = jnp.sum(ex, axis=0, keepdims=True)
        a = ex * (1.0 / jnp.maximum(denom, 1e-20))
        for jtl in range(Wj // 128):
            att_ref[0, :, jtl * H + h, :] = a[:, jtl * 128:(jtl + 1) * 128]
        outs.append(jnp.dot(a, fblk_h, preferred_element_type=jnp.float32))

    partial = jnp.concatenate(outs, axis=1)

    @pl.when(jb == 0)
    def _():
        out_ref[0] = partial

    @pl.when(jb > 0)
    def _():
        out_ref[0] += partial


def kernel(node_feat, adj_matrix, W, attn_l, attn_r):
    B, N, in_dim = node_feat.shape
    H, D = attn_l.shape[1], attn_l.shape[2]
    al = attn_l.reshape(H, D)
    ar = attn_r.reshape(H, D)
    JT = N // 128
    WJ = 256
    JB = N // WJ

    out, att = pl.pallas_call(
        _gat_fused,
        grid=(B, JB),
        in_specs=[
            pl.BlockSpec((1, N, in_dim), lambda b, jb: (b, 0, 0)),
            pl.BlockSpec((1, N, WJ), lambda b, jb: (b, 0, jb)),
            pl.BlockSpec((in_dim, H * D), lambda b, jb: (0, 0)),
            pl.BlockSpec((H, D), lambda b, jb: (0, 0)),
            pl.BlockSpec((H, D), lambda b, jb: (0, 0)),
        ],
        out_specs=[
            pl.BlockSpec((1, N, H * D), lambda b, jb: (b, 0, 0)),
            pl.BlockSpec((1, N, (WJ // 128) * H, 128), lambda b, jb: (b, 0, jb, 0)),
        ],
        out_shape=[
            jax.ShapeDtypeStruct((B, N, H * D), jnp.float32),
            jax.ShapeDtypeStruct((B, N, JT * H, 128), jnp.float32),
        ],
        scratch_shapes=[pltpu.VMEM((N, H * D), jnp.float32)],
    )(node_feat, adj_matrix, W, al, ar)
    attention = (att.reshape(B, N, JT, H, 128)
                 .transpose(0, 1, 2, 4, 3)
                 .reshape(B, N, N, H))
    return out, attention


# speedup vs baseline: 1.0263x; 1.0263x over previous
import jax
import jax.numpy as jnp
from jax.experimental import pallas as pl
from jax.experimental.pallas import tpu as pltpu


def _gat_fused(nf_ref, adj_ref, w_ref, al_ref, ar_ref, out_ref, att_ref,
               feat_ref, el_ref, er_ref):
    H, D = al_ref.shape
    JB, Wj, HD = feat_ref.shape
    jb = pl.program_id(1)

    @pl.when(jb == 0)
    def _():
        f = jnp.dot(nf_ref[0], w_ref[...], preferred_element_type=jnp.float32)
        feat_ref[...] = f.reshape(JB, Wj, HD)
        for h in range(H):
            f_h = f[:, h * D:(h + 1) * D]
            al_h = al_ref[h, :].reshape(1, D)
            ar_h = ar_ref[h, :].reshape(1, D)
            el_ref[:, h:h + 1] = jax.lax.dot_general(
                f_h, al_h, (((1,), (1,)), ((), ())),
                preferred_element_type=jnp.float32)
            er_h = jax.lax.dot_general(
                ar_h, f_h, (((1,), (1,)), ((), ())),
                preferred_element_type=jnp.float32)
            er_ref[:, h:h + 1, :] = er_h.reshape(JB, 1, Wj)

    mask = adj_ref[0] > 0
    fblk = feat_ref[jb]
    outs = []
    for h in range(H):
        fblk_h = fblk[:, h * D:(h + 1) * D]
        el = el_ref[:, h:h + 1]
        er = er_ref[jb, h:h + 1, :]
        s = el + er
        s = jnp.maximum(s, 0.2 * s)
        m = jnp.max(jnp.where(mask, s, -1e30), axis=0, keepdims=True)
        ex = jnp.where(mask, jnp.exp(s - m), 0.0)
        denom = jnp.sum(ex, axis=0, keepdims=True)
        a = ex * (1.0 / jnp.maximum(denom, 1e-20))
        for jtl in range(Wj // 128):
            att_ref[0, :, jtl * H + h, :] = a[:, jtl * 128:(jtl + 1) * 128]
        outs.append(jnp.dot(a, fblk_h, preferred_element_type=jnp.float32))

    partial = jnp.concatenate(outs, axis=1)

    @pl.when(jb == 0)
    def _():
        out_ref[0] = partial

    @pl.when(jb > 0)
    def _():
        out_ref[0] += partial


def kernel(node_feat, adj_matrix, W, attn_l, attn_r):
    B, N, in_dim = node_feat.shape
    H, D = attn_l.shape[1], attn_l.shape[2]
    al = attn_l.reshape(H, D)
    ar = attn_r.reshape(H, D)
    JT = N // 128
    WJ = 256
    JB = N // WJ

    out, att = pl.pallas_call(
        _gat_fused,
        grid=(B, JB),
        in_specs=[
            pl.BlockSpec((1, N, in_dim), lambda b, jb: (b, 0, 0)),
            pl.BlockSpec((1, N, WJ), lambda b, jb: (b, 0, jb)),
            pl.BlockSpec((in_dim, H * D), lambda b, jb: (0, 0)),
            pl.BlockSpec((H, D), lambda b, jb: (0, 0)),
            pl.BlockSpec((H, D), lambda b, jb: (0, 0)),
        ],
        out_specs=[
            pl.BlockSpec((1, N, H * D), lambda b, jb: (b, 0, 0)),
            pl.BlockSpec((1, N, (WJ // 128) * H, 128),
                         lambda b, jb: (b, 0, jb, 0)),
        ],
        out_shape=[
            jax.ShapeDtypeStruct((B, N, H * D), jnp.float32),
            jax.ShapeDtypeStruct((B, N, JT * H, 128), jnp.float32),
        ],
        scratch_shapes=[
            pltpu.VMEM((JB, WJ, H * D), jnp.float32),
            pltpu.VMEM((N, 8), jnp.float32),
            pltpu.VMEM((JB, 8, WJ), jnp.float32),
        ],
    )(node_feat, adj_matrix, W, al, ar)
    attention = (att.reshape(B, N, JT, H, 128)
                 .transpose(0, 1, 2, 4, 3)
                 .reshape(B, N, N, H))
    return out, attention


# speedup vs baseline: 1.0814x; 1.0536x over previous
import jax
import jax.numpy as jnp
from jax.experimental import pallas as pl
from jax.experimental.pallas import tpu as pltpu


def _gat_fused(nf_ref, adj_ref, w_ref, al_ref, ar_ref, out_ref, att_ref,
               feat_ref, el_ref, er_ref):
    H, D = al_ref.shape
    JB, Wj, HD = feat_ref.shape
    jb = pl.program_id(1)

    @pl.when(jb == 0)
    def _():
        f = jnp.dot(nf_ref[0], w_ref[...], preferred_element_type=jnp.float32)
        feat_ref[...] = f.reshape(JB, Wj, HD)
        for h in range(H):
            f_h = f[:, h * D:(h + 1) * D]
            al_h = al_ref[h, :].reshape(1, D)
            ar_h = ar_ref[h, :].reshape(1, D)
            el_ref[:, h:h + 1] = jax.lax.dot_general(
                f_h, al_h, (((1,), (1,)), ((), ())),
                preferred_element_type=jnp.float32)
            er_h = jax.lax.dot_general(
                ar_h, f_h, (((1,), (1,)), ((), ())),
                preferred_element_type=jnp.float32)
            er_ref[:, h:h + 1, :] = er_h.reshape(JB, 1, Wj)

    mask = adj_ref[0] > 0
    fblk = feat_ref[jb]
    outs = []
    for h in range(H):
        fblk_h = fblk[:, h * D:(h + 1) * D]
        el = el_ref[:, h:h + 1]
        er = er_ref[jb, h:h + 1, :]
        s = el + er
        s = jnp.maximum(s, 0.2 * s)
        sm = jnp.where(mask, s, -1e30)
        m = jnp.max(sm, axis=0, keepdims=True)
        ex = jnp.exp(sm - m)
        denom = jnp.sum(ex, axis=0, keepdims=True)
        recip = jnp.where(m > -9e29, 1.0 / jnp.maximum(denom, 1e-20), 0.0)
        a = ex * recip
        for jtl in range(Wj // 128):
            att_ref[0, :, jtl * H + h, :] = a[:, jtl * 128:(jtl + 1) * 128]
        outs.append(jnp.dot(a, fblk_h, preferred_element_type=jnp.float32))

    partial = jnp.concatenate(outs, axis=1)

    @pl.when(jb == 0)
    def _():
        out_ref[0] = partial

    @pl.when(jb > 0)
    def _():
        out_ref[0] += partial


def kernel(node_feat, adj_matrix, W, attn_l, attn_r):
    B, N, in_dim = node_feat.shape
    H, D = attn_l.shape[1], attn_l.shape[2]
    al = attn_l.reshape(H, D)
    ar = attn_r.reshape(H, D)
    JT = N // 128
    WJ = 256
    JB = N // WJ

    out, att = pl.pallas_call(
        _gat_fused,
        grid=(B, JB),
        in_specs=[
            pl.BlockSpec((1, N, in_dim), lambda b, jb: (b, 0, 0)),
            pl.BlockSpec((1, N, WJ), lambda b, jb: (b, 0, jb)),
            pl.BlockSpec((in_dim, H * D), lambda b, jb: (0, 0)),
            pl.BlockSpec((H, D), lambda b, jb: (0, 0)),
            pl.BlockSpec((H, D), lambda b, jb: (0, 0)),
        ],
        out_specs=[
            pl.BlockSpec((1, N, H * D), lambda b, jb: (b, 0, 0)),
            pl.BlockSpec((1, N, (WJ // 128) * H, 128),
                         lambda b, jb: (b, 0, jb, 0)),
        ],
        out_shape=[
            jax.ShapeDtypeStruct((B, N, H * D), jnp.float32),
            jax.ShapeDtypeStruct((B, N, JT * H, 128), jnp.float32),
        ],
        scratch_shapes=[
            pltpu.VMEM((JB, WJ, H * D), jnp.float32),
            pltpu.VMEM((N, 8), jnp.float32),
            pltpu.VMEM((JB, 8, WJ), jnp.float32),
        ],
    )(node_feat, adj_matrix, W, al, ar)
    attention = (att.reshape(B, N, JT, H, 128)
                 .transpose(0, 1, 2, 4, 3)
                 .reshape(B, N, N, H))
    return out, attention


# speedup vs baseline: 1.1215x; 1.0371x over previous
import jax
import jax.numpy as jnp
import numpy as np
from jax.experimental import pallas as pl
from jax.experimental.pallas import tpu as pltpu


def _gat_fused(nf_ref, adj_ref, w_ref, al_ref, ar_ref, out_ref, att_ref,
               feat_ref, el_ref, er_ref):
    H, D = al_ref.shape
    JB, Wj, HD = feat_ref.shape
    jb = pl.program_id(1)

    @pl.when(jb == 0)
    def _():
        f = jnp.dot(nf_ref[0], w_ref[...], preferred_element_type=jnp.float32)
        feat_ref[...] = f.reshape(JB, Wj, HD)
        for h in range(H):
            f_h = f[:, h * D:(h + 1) * D]
            al_h = al_ref[h, :].reshape(1, D)
            ar_h = ar_ref[h, :].reshape(1, D)
            el_h = jax.lax.dot_general(
                f_h, al_h, (((1,), (1,)), ((), ())),
                preferred_element_type=jnp.float32)
            el_ref[:, h:h + 1] = el_h * np.float32(np.log2(np.e))
            er_h = jax.lax.dot_general(
                ar_h, f_h, (((1,), (1,)), ((), ())),
                preferred_element_type=jnp.float32)
            er_ref[:, h:h + 1, :] = (er_h * np.float32(np.log2(np.e))).reshape(JB, 1, Wj)

    mask = adj_ref[0] > 0
    fblk = feat_ref[jb]
    outs = []
    for h in range(H):
        fblk_h = fblk[:, h * D:(h + 1) * D]
        el = el_ref[:, h:h + 1]
        er = er_ref[jb, h:h + 1, :]
        s = el + er
        s = jnp.maximum(s, 0.2 * s)
        sm = jnp.where(mask, s, -1e30)
        m = jnp.max(sm, axis=0, keepdims=True)
        ex = jnp.exp2(sm - m)
        denom = jnp.sum(ex, axis=0, keepdims=True)
        recip = jnp.where(m > -9e29, 1.0 / jnp.maximum(denom, 1e-20), 0.0)
        a = ex * recip
        for jtl in range(Wj // 128):
            att_ref[0, :, jtl * H + h, :] = a[:, jtl * 128:(jtl + 1) * 128]
        outs.append(jnp.dot(a, fblk_h, preferred_element_type=jnp.float32))

    partial = jnp.concatenate(outs, axis=1)

    @pl.when(jb == 0)
    def _():
        out_ref[0] = partial

    @pl.when(jb > 0)
    def _():
        out_ref[0] += partial


def kernel(node_feat, adj_matrix, W, attn_l, attn_r):
    B, N, in_dim = node_feat.shape
    H, D = attn_l.shape[1], attn_l.shape[2]
    al = attn_l.reshape(H, D)
    ar = attn_r.reshape(H, D)
    JT = N // 128
    WJ = 256
    JB = N // WJ

    out, att = pl.pallas_call(
        _gat_fused,
        grid=(B, JB),
        in_specs=[
            pl.BlockSpec((1, N, in_dim), lambda b, jb: (b, 0, 0)),
            pl.BlockSpec((1, N, WJ), lambda b, jb: (b, 0, jb)),
            pl.BlockSpec((in_dim, H * D), lambda b, jb: (0, 0)),
            pl.BlockSpec((H, D), lambda b, jb: (0, 0)),
            pl.BlockSpec((H, D), lambda b, jb: (0, 0)),
        ],
        out_specs=[
            pl.BlockSpec((1, N, H * D), lambda b, jb: (b, 0, 0)),
            pl.BlockSpec((1, N, (WJ // 128) * H, 128),
                         lambda b, jb: (b, 0, jb, 0)),
        ],
        out_shape=[
            jax.ShapeDtypeStruct((B, N, H * D), jnp.float32),
            jax.ShapeDtypeStruct((B, N, JT * H, 128), jnp.float32),
        ],
        scratch_shapes=[
            pltpu.VMEM((JB, WJ, H * D), jnp.float32),
            pltpu.VMEM((N, 8), jnp.float32),
            pltpu.VMEM((JB, 8, WJ), jnp.float32),
        ],
    )(node_feat, adj_matrix, W, al, ar)
    attention = (att.reshape(B, N, JT, H, 128)
                 .transpose(0, 1, 2, 4, 3)
                 .reshape(B, N, N, H))
    return out, attention
